# baseline (speedup 1.0000x reference)
import functools

import numpy as np
import jax
import jax.numpy as jnp
from jax import lax
from jax.experimental import pallas as pl
from jax.experimental.pallas import tpu as pltpu
from jax.experimental.pallas import tpu_sc as plsc

_NUM_CLASSES = 80
_NUM_FG = 128
_NUM_BG = 384
_P = 20000
_G = 64
_NW = 16
_PP = 20480
_CHUNK = _PP // _NW
_VPC = _CHUNK // 16
_ROWS = _PP // 128
_TC_TILE = 16
_DUMP = _NUM_FG + _NUM_BG


def _rotl(x, r):
    return ((x << np.uint32(r)) | (x >> np.uint32(32 - r))) & np.uint32(0xFFFFFFFF)


def _threefry2x32(key0, key1, x0, x1):
    ks0 = np.uint32(key0)
    ks1 = np.uint32(key1)
    ks2 = ks0 ^ ks1 ^ np.uint32(0x1BD11BDA)
    x0 = (x0 + ks0).astype(np.uint32)
    x1 = (x1 + ks1).astype(np.uint32)
    rots = ([13, 15, 26, 6], [17, 29, 16, 24])
    inj = [(ks1, ks2), (ks2, ks0), (ks0, ks1), (ks1, ks2), (ks2, ks0)]
    for i in range(5):
        for r in rots[i % 2]:
            x0 = (x0 + x1).astype(np.uint32)
            x1 = _rotl(x1, r)
            x1 = x1 ^ x0
        a, b = inj[i]
        x0 = (x0 + a).astype(np.uint32)
        x1 = (x1 + b + np.uint32(i + 1)).astype(np.uint32)
    return x0, x1


def _fry_uniform(key, n):
    o0, o1 = _threefry2x32(key[0], key[1],
                           np.zeros(n, np.uint32), np.arange(n, dtype=np.uint32))
    bits = o0 ^ o1
    return ((bits >> np.uint32(9)) | np.uint32(0x3F800000)).view(np.float32) \
        - np.float32(1.0)


@functools.cache
def _perm_consts():
    o0, o1 = _threefry2x32(np.uint32(0), np.uint32(1),
                           np.zeros(2, np.uint32), np.arange(2, dtype=np.uint32))
    u1 = _fry_uniform((o0[0], o1[0]), _P)
    u2 = _fry_uniform((o0[1], o1[1]), _P)
    keys = [
        (np.float32(1.0) + u1),
        u1,
        (np.float32(1.0) + u2),
        u2,
    ]
    pad = np.arange(_P, _PP, dtype=np.int32)
    return [np.concatenate([np.argsort(-k, kind="stable").astype(np.int32), pad])
            for k in keys]


def _iou_body(pbt_ref, gt_ref, vals_ref, idxs_ref, lab_ref):
    x0 = pbt_ref[0]
    y0 = pbt_ref[1]
    x1 = pbt_ref[2]
    y1 = pbt_ref[3]
    area2 = (x1 - x0) * (y1 - y0)

    def body(g, carry):
        vals, idxs = carry
        gx0 = gt_ref[g, 0]
        gy0 = gt_ref[g, 1]
        gx1 = gt_ref[g, 2]
        gy1 = gt_ref[g, 3]
        a1 = (gx1 - gx0) * (gy1 - gy0)
        w = jnp.maximum(jnp.minimum(gx1, x1) - jnp.maximum(gx0, x0), 0.0)
        h = jnp.maximum(jnp.minimum(gy1, y1) - jnp.maximum(gy0, y0), 0.0)
        inter = w * h
        iou = jnp.where(inter > 0, inter / (a1 + area2 - inter), 0.0)
        better = iou > vals
        return jnp.where(better, iou, vals), jnp.where(better, g, idxs)

    vals0 = jnp.zeros(x0.shape, jnp.float32)
    idxs0 = jnp.zeros(x0.shape, jnp.int32)
    vals, idxs = lax.fori_loop(0, _G, body, (vals0, idxs0))
    vals_ref[...] = vals
    idxs_ref[...] = idxs
    r = lax.broadcasted_iota(jnp.int32, x0.shape, 0)
    c = lax.broadcasted_iota(jnp.int32, x0.shape, 1)
    gidx = (pl.program_id(0) * _TC_TILE + r) * 128 + c
    fg = (vals >= 0.5).astype(jnp.int32)
    lab_ref[...] = jnp.where(gidx < _P, fg, -1)


def _iou_call(pbt, gt_boxes):
    n_steps = _ROWS // _TC_TILE
    return pl.pallas_call(
        _iou_body,
        grid=(n_steps,),
        in_specs=[
            pl.BlockSpec((4, _TC_TILE, 128), lambda i: (0, i, 0)),
            pl.BlockSpec(memory_space=pltpu.SMEM),
        ],
        out_specs=[
            pl.BlockSpec((_TC_TILE, 128), lambda i: (i, 0)),
            pl.BlockSpec((_TC_TILE, 128), lambda i: (i, 0)),
            pl.BlockSpec((_TC_TILE, 128), lambda i: (i, 0)),
        ],
        out_shape=[
            jax.ShapeDtypeStruct((_ROWS, 128), jnp.float32),
            jax.ShapeDtypeStruct((_ROWS, 128), jnp.int32),
            jax.ShapeDtypeStruct((_ROWS, 128), jnp.int32),
        ],
    )(pbt, gt_boxes)


_STREAMS = (
    (1, _NUM_FG, 0, False),
    (0, _NUM_FG, 0, True),
    (0, _NUM_BG, _NUM_FG, False),
    (1, _NUM_BG, _NUM_FG, True),
)


def _lane(v, s):
    return jnp.sum(jnp.where(lax.iota(jnp.int32, 16) == s, v, 0))


def _sc_body(vals_hbm, idxs_hbm, lab_hbm, p0_hbm, p1_hbm, p2_hbm, p3_hbm,
             pf0_hbm, pf1_hbm, pf2_hbm, pf3_hbm,
             pb0_hbm, pb1_hbm, pb2_hbm, pb3_hbm,
             gt0_hbm, gt1_hbm, gt2_hbm, gt3_hbm, gtc_hbm,
             out_hbm, cls_hbm,
             perm0_v, perm1_v, perm2_v, perm3_v,
             perm2d0, perm2d1, perm2d2, perm2d3,
             labs0_v, labs1_v, labs2_v, labs3_v,
             lst0, lst1, lst2, lst3,
             pos0, pos1, pos2, pos3,
             cnt_v, allc_v,
             sidx_v, sp0_v, sp1_v, sp2_v, sp3_v, sv_v, mi_v,
             sg0_v, sg1_v, sg2_v, sg3_v, sc_v, outbuf, cls_v,
             counts_sh, sidx_sh, sem):
    wid = lax.axis_index("s")
    lane = lax.iota(jnp.int32, 16)

    perm_hbms = (p0_hbm, p1_hbm, p2_hbm, p3_hbm)
    permf_hbms = (pf0_hbm, pf1_hbm, pf2_hbm, pf3_hbm)
    perm_vs = (perm0_v, perm1_v, perm2_v, perm3_v)
    perm2ds = (perm2d0, perm2d1, perm2d2, perm2d3)
    labs_vs = (labs0_v, labs1_v, labs2_v, labs3_v)
    lsts = (lst0, lst1, lst2, lst3)
    poss = (pos0, pos1, pos2, pos3)
    rows = _CHUNK // 128
    descs = []
    for s in range(4):
        descs.append(pltpu.async_copy(
            perm_hbms[s].at[pl.ds(wid * rows, rows)], perm2ds[s], sem))
        descs.append(pltpu.async_copy(
            permf_hbms[s].at[pl.ds(wid * _CHUNK, _CHUNK)], perm_vs[s], sem))
    for d in descs:
        d.wait()
    descs = []
    for s in range(4):
        for k in range(rows):
            descs.append(pltpu.async_copy(
                lab_hbm.at[perm2ds[s].at[k]],
                labs_vs[s].at[pl.ds(k * 128, 128)], sem))
    for d in descs:
        d.wait()

    def _first(v):
        return lax.squeeze(lax.slice(v, (0,), (1,)), (0,))

    def cbody(j, cnts):
        news = list(cnts)
        for u in range(2):
            for s, (target, cap, _, _) in enumerate(_STREAMS):
                sl = pl.ds((j * 2 + u) * 16, 16)
                idxv = perm_vs[s][sl]
                labv = labs_vs[s][sl]
                m = labv == target
                off = jnp.minimum(news[s], cap)
                plsc.store_compressed(lsts[s].at[pl.ds(off, 16)], idxv, mask=m)
                pc = _first(plsc.all_reduce_population_count(m))
                news[s] = news[s] + pc
        return tuple(news)

    counts = list(lax.fori_loop(0, _VPC // 2, cbody,
                                (jnp.int32(0),) * 4))

    cvec = jnp.zeros((16,), jnp.int32)
    for s in range(4):
        cvec = cvec + jnp.where(lane == s, counts[s], 0)
    cnt_v[...] = cvec
    pltpu.sync_copy(cnt_v, counts_sh.at[wid])
    plsc.subcore_barrier()
    pltpu.sync_copy(counts_sh, allc_v)

    base_vec = jnp.zeros((16,), jnp.int32)
    tot_vec = jnp.zeros((16,), jnp.int32)
    for w2 in range(_NW):
        row = allc_v[w2]
        base_vec = base_vec + jnp.where(jnp.int32(w2) < wid, row, 0)
        tot_vec = tot_vec + row
    bases = [_lane(base_vec, s) for s in range(4)]
    tot_fg = _lane(tot_vec, 0)
    tot_bg = _lane(tot_vec, 2)
    tots = (None, tot_fg, None, tot_bg)

    for s, (_, cap, out_off, is_fill) in enumerate(_STREAMS):
        pos = poss[s]
        base = bases[s]
        c_self = counts[s]
        for k in range(cap // 128):
            for jj in range(8):
                jvec = k * 128 + jj * 16 + lane
                gpos = base + jvec
                if is_fill:
                    gpos = tots[s] + gpos
                m = (jvec < c_self) & (gpos < cap)
                pos[k, pl.ds(jj * 16, 16)] = jnp.where(m, gpos + out_off, _DUMP)
    descs = []
    for s, (_, cap, _, _) in enumerate(_STREAMS):
        for k in range(cap // 128):
            descs.append(pltpu.async_copy(lsts[s].at[pl.ds(k * 128, 128)],
                                          sidx_sh.at[poss[s].at[k]], sem))
    for d in descs:
        d.wait()
    plsc.subcore_barrier()

    nslot = (_NUM_FG + _NUM_BG) // _NW
    pltpu.sync_copy(sidx_sh.at[pl.ds(wid * nslot, nslot)], sidx_v)
    sps = (sp0_v, sp1_v, sp2_v, sp3_v)
    sgs = (sg0_v, sg1_v, sg2_v, sg3_v)
    descs = [pltpu.async_copy(vals_hbm.at[sidx_v], sv_v, sem),
             pltpu.async_copy(idxs_hbm.at[sidx_v], mi_v, sem)]
    for c, t in enumerate((pb0_hbm, pb1_hbm, pb2_hbm, pb3_hbm)):
        descs.append(pltpu.async_copy(t.at[sidx_v], sps[c], sem))
    for d in descs:
        d.wait()
    descs = [pltpu.async_copy(gtc_hbm.at[mi_v], sc_v, sem)]
    for c, t in enumerate((gt0_hbm, gt1_hbm, gt2_hbm, gt3_hbm)):
        descs.append(pltpu.async_copy(t.at[mi_v], sgs[c], sem))
    for d in descs:
        d.wait()

    for k in range(nslot // 16):
        rvec = k * 16 + lane
        for c in range(4):
            csplat = jnp.broadcast_to(jnp.int32(c), (16,))
            plsc.store_scatter(outbuf, [rvec, csplat], sps[c][pl.ds(k * 16, 16)])
            plsc.store_scatter(outbuf, [rvec, csplat + 4], sgs[c][pl.ds(k * 16, 16)])
        svv = sv_v[pl.ds(k * 16, 16)]
        plsc.store_scatter(outbuf, [rvec, jnp.broadcast_to(jnp.int32(8), (16,))], svv)
        gtcv = sc_v[pl.ds(k * 16, 16)]
        cls_v[pl.ds(k * 16, 16)] = jnp.where(svv >= 0.5, gtcv, _NUM_CLASSES)

    pltpu.sync_copy(outbuf, out_hbm.at[pl.ds(wid * nslot, nslot)])
    pltpu.sync_copy(cls_v, cls_hbm.at[pl.ds(wid * nslot, nslot)])


def _sc_call():
    mesh = plsc.VectorSubcoreMesh(core_axis_name="c", subcore_axis_name="s",
                                  num_cores=1)
    nslot = (_NUM_FG + _NUM_BG) // _NW
    return pl.kernel(
        _sc_body,
        out_type=[
            jax.ShapeDtypeStruct((_NUM_FG + _NUM_BG, 9), jnp.float32),
            jax.ShapeDtypeStruct((_NUM_FG + _NUM_BG,), jnp.int32),
        ],
        mesh=mesh,
        compiler_params=pltpu.CompilerParams(needs_layout_passes=False,
                                             use_tc_tiling_on_sc=False),
        scratch_types=[
            pltpu.VMEM((_CHUNK,), jnp.int32),
            pltpu.VMEM((_CHUNK,), jnp.int32),
            pltpu.VMEM((_CHUNK,), jnp.int32),
            pltpu.VMEM((_CHUNK,), jnp.int32),
            pltpu.VMEM((_CHUNK // 128, 128), jnp.int32),
            pltpu.VMEM((_CHUNK // 128, 128), jnp.int32),
            pltpu.VMEM((_CHUNK // 128, 128), jnp.int32),
            pltpu.VMEM((_CHUNK // 128, 128), jnp.int32),
            pltpu.VMEM((_CHUNK,), jnp.int32),
            pltpu.VMEM((_CHUNK,), jnp.int32),
            pltpu.VMEM((_CHUNK,), jnp.int32),
            pltpu.VMEM((_CHUNK,), jnp.int32),
            pltpu.VMEM((_NUM_FG + 16,), jnp.int32),
            pltpu.VMEM((_NUM_FG + 16,), jnp.int32),
            pltpu.VMEM((_NUM_BG + 16,), jnp.int32),
            pltpu.VMEM((_NUM_BG + 16,), jnp.int32),
            pltpu.VMEM((1, 128), jnp.int32),
            pltpu.VMEM((1, 128), jnp.int32),
            pltpu.VMEM((3, 128), jnp.int32),
            pltpu.VMEM((3, 128), jnp.int32),
            pltpu.VMEM((16,), jnp.int32),
            pltpu.VMEM((_NW, 16), jnp.int32),
            pltpu.VMEM((nslot,), jnp.int32),
            pltpu.VMEM((nslot,), jnp.float32),
            pltpu.VMEM((nslot,), jnp.float32),
            pltpu.VMEM((nslot,), jnp.float32),
            pltpu.VMEM((nslot,), jnp.float32),
            pltpu.VMEM((nslot,), jnp.float32),
            pltpu.VMEM((nslot,), jnp.int32),
            pltpu.VMEM((nslot,), jnp.float32),
            pltpu.VMEM((nslot,), jnp.float32),
            pltpu.VMEM((nslot,), jnp.float32),
            pltpu.VMEM((nslot,), jnp.float32),
            pltpu.VMEM((nslot,), jnp.int32),
            pltpu.VMEM((nslot, 9), jnp.float32),
            pltpu.VMEM((nslot,), jnp.int32),
            pltpu.VMEM_SHARED((_NW, 16), jnp.int32),
            pltpu.VMEM_SHARED((_DUMP + 16,), jnp.int32),
            pltpu.SemaphoreType.DMA,
        ],
    )


def kernel(proposal_boxes, gt_boxes, gt_classes):
    perms = [jnp.asarray(p) for p in _perm_consts()]
    pb_pad = jnp.concatenate(
        [proposal_boxes, jnp.zeros((_PP - _P, 4), jnp.float32)], axis=0)
    pbt4 = pb_pad.T
    pbt = pbt4.reshape(4, _ROWS, 128)
    vals, idxs, lab = _iou_call(pbt, gt_boxes)
    gtt = gt_boxes.T
    out, cls = _sc_call()(
        vals.reshape(_PP), idxs.reshape(_PP), lab.reshape(_PP),
        perms[0].reshape(_PP // 128, 128), perms[1].reshape(_PP // 128, 128),
        perms[2].reshape(_PP // 128, 128), perms[3].reshape(_PP // 128, 128),
        perms[0], perms[1], perms[2], perms[3],
        pbt4[0], pbt4[1], pbt4[2], pbt4[3],
        gtt[0], gtt[1], gtt[2], gtt[3], gt_classes.astype(jnp.int32))
    return out, cls


# speedup vs baseline: 1.2384x; 1.2384x over previous
import functools

import numpy as np
import jax
import jax.numpy as jnp
from jax import lax
from jax.experimental import pallas as pl
from jax.experimental.pallas import tpu as pltpu
from jax.experimental.pallas import tpu_sc as plsc

_NUM_CLASSES = 80
_NUM_FG = 128
_NUM_BG = 384
_P = 20000
_G = 64
_NW = 16
_PP = 20480
_CHUNK = _PP // _NW
_VPC = _CHUNK // 16
_ROWS = _PP // 128
_TC_TILE = 16
_DUMP = _NUM_FG + _NUM_BG


def _rotl(x, r):
    return ((x << np.uint32(r)) | (x >> np.uint32(32 - r))) & np.uint32(0xFFFFFFFF)


def _threefry2x32(key0, key1, x0, x1):
    ks0 = np.uint32(key0)
    ks1 = np.uint32(key1)
    ks2 = ks0 ^ ks1 ^ np.uint32(0x1BD11BDA)
    x0 = (x0 + ks0).astype(np.uint32)
    x1 = (x1 + ks1).astype(np.uint32)
    rots = ([13, 15, 26, 6], [17, 29, 16, 24])
    inj = [(ks1, ks2), (ks2, ks0), (ks0, ks1), (ks1, ks2), (ks2, ks0)]
    for i in range(5):
        for r in rots[i % 2]:
            x0 = (x0 + x1).astype(np.uint32)
            x1 = _rotl(x1, r)
            x1 = x1 ^ x0
        a, b = inj[i]
        x0 = (x0 + a).astype(np.uint32)
        x1 = (x1 + b + np.uint32(i + 1)).astype(np.uint32)
    return x0, x1


def _fry_uniform(key, n):
    o0, o1 = _threefry2x32(key[0], key[1],
                           np.zeros(n, np.uint32), np.arange(n, dtype=np.uint32))
    bits = o0 ^ o1
    return ((bits >> np.uint32(9)) | np.uint32(0x3F800000)).view(np.float32) \
        - np.float32(1.0)


@functools.cache
def _perm_consts():
    o0, o1 = _threefry2x32(np.uint32(0), np.uint32(1),
                           np.zeros(2, np.uint32), np.arange(2, dtype=np.uint32))
    u1 = _fry_uniform((o0[0], o1[0]), _P)
    u2 = _fry_uniform((o0[1], o1[1]), _P)
    keys = [
        (np.float32(1.0) + u1),
        u1,
        (np.float32(1.0) + u2),
        u2,
    ]
    pad = np.arange(_P, _PP, dtype=np.int32)
    return [np.concatenate([np.argsort(-k, kind="stable").astype(np.int32), pad])
            for k in keys]


def _iou_body(pbt_ref, gt_ref, vals_ref, idxs_ref, lab_ref):
    x0 = pbt_ref[0]
    y0 = pbt_ref[1]
    x1 = pbt_ref[2]
    y1 = pbt_ref[3]
    area2 = (x1 - x0) * (y1 - y0)

    def body(g, carry):
        vals, idxs = carry
        gx0 = gt_ref[g, 0]
        gy0 = gt_ref[g, 1]
        gx1 = gt_ref[g, 2]
        gy1 = gt_ref[g, 3]
        a1 = (gx1 - gx0) * (gy1 - gy0)
        w = jnp.maximum(jnp.minimum(gx1, x1) - jnp.maximum(gx0, x0), 0.0)
        h = jnp.maximum(jnp.minimum(gy1, y1) - jnp.maximum(gy0, y0), 0.0)
        inter = w * h
        iou = jnp.where(inter > 0, inter / (a1 + area2 - inter), 0.0)
        better = iou > vals
        return jnp.where(better, iou, vals), jnp.where(better, g, idxs)

    vals0 = jnp.zeros(x0.shape, jnp.float32)
    idxs0 = jnp.zeros(x0.shape, jnp.int32)
    vals, idxs = lax.fori_loop(0, _G, body, (vals0, idxs0))
    vals_ref[...] = vals
    idxs_ref[...] = idxs
    r = lax.broadcasted_iota(jnp.int32, x0.shape, 0)
    c = lax.broadcasted_iota(jnp.int32, x0.shape, 1)
    gidx = (pl.program_id(0) * _TC_TILE + r) * 128 + c
    fg = (vals >= 0.5).astype(jnp.int32)
    lab_ref[...] = jnp.where(gidx < _P, fg, -1)


def _iou_call(pbt, gt_boxes):
    n_steps = _ROWS // _TC_TILE
    return pl.pallas_call(
        _iou_body,
        grid=(n_steps,),
        in_specs=[
            pl.BlockSpec((4, _TC_TILE, 128), lambda i: (0, i, 0)),
            pl.BlockSpec(memory_space=pltpu.SMEM),
        ],
        out_specs=[
            pl.BlockSpec((_TC_TILE, 128), lambda i: (i, 0)),
            pl.BlockSpec((_TC_TILE, 128), lambda i: (i, 0)),
            pl.BlockSpec((_TC_TILE, 128), lambda i: (i, 0)),
        ],
        out_shape=[
            jax.ShapeDtypeStruct((_ROWS, 128), jnp.float32),
            jax.ShapeDtypeStruct((_ROWS, 128), jnp.int32),
            jax.ShapeDtypeStruct((_ROWS, 128), jnp.int32),
        ],
    )(pbt, gt_boxes)


_STREAMS = (
    (1, _NUM_FG, 0),
    (0, _NUM_FG, 0),
    (0, _NUM_BG, _NUM_FG),
    (1, _NUM_BG, _NUM_FG),
)


def _lane(v, s):
    return jnp.sum(jnp.where(lax.iota(jnp.int32, 16) == s, v, 0))


def _first(v):
    return lax.squeeze(lax.slice(v, (0,), (1,)), (0,))


def _sc_body(vals_hbm, idxs_hbm, lab_hbm, p0_hbm, p1_hbm, p2_hbm, p3_hbm,
             pb0_hbm, pb1_hbm, pb2_hbm, pb3_hbm,
             gt0_hbm, gt1_hbm, gt2_hbm, gt3_hbm, gtc_hbm,
             out_hbm, cls_hbm,
             label_v, perm0_v, perm1_v, perm2_v, perm3_v,
             lst0, lst1, lst2, lst3,
             pos0, pos1, pos2, pos3,
             cnt_v, allc_v,
             sidx_v, sp0_v, sp1_v, sp2_v, sp3_v, sv_v, mi_v,
             sg0_v, sg1_v, sg2_v, sg3_v, sc_v, outbuf, cls_v,
             counts_sh, sidx_sh, sem):
    wid = lax.axis_index("s")
    lane = lax.iota(jnp.int32, 16)

    perm_hbms = (p0_hbm, p1_hbm, p2_hbm, p3_hbm)
    perm_vs = (perm0_v, perm1_v, perm2_v, perm3_v)
    lsts = (lst0, lst1, lst2, lst3)
    poss = (pos0, pos1, pos2, pos3)
    descs = [pltpu.async_copy(lab_hbm, label_v, sem)]
    for s in range(4):
        descs.append(pltpu.async_copy(
            perm_hbms[s].at[pl.ds(wid * _CHUNK, _CHUNK)], perm_vs[s], sem))
    for d in descs:
        d.wait()

    def scan_one(s, cnt, j):
        target, cap, _ = _STREAMS[s]
        idxv = perm_vs[s][pl.ds(j * 16, 16)]
        labv = plsc.load_gather(label_v, [idxv])
        m = labv == target
        off = jnp.minimum(cnt, cap)
        plsc.store_compressed(lsts[s].at[pl.ds(off, 16)], idxv, mask=m)
        return cnt + _first(plsc.all_reduce_population_count(m))

    def cbody(j, cnts):
        cA1, cA2 = cnts
        for u in range(2):
            cA1 = scan_one(0, cA1, j * 2 + u)
            cA2 = scan_one(2, cA2, j * 2 + u)
        return cA1, cA2

    cA1, cA2 = lax.fori_loop(0, _VPC // 2, cbody, (jnp.int32(0),) * 2)

    cnt_v[...] = jnp.where(lane == 0, cA1, 0) + jnp.where(lane == 2, cA2, 0)
    pltpu.sync_copy(cnt_v, counts_sh.at[wid])
    plsc.subcore_barrier()
    pltpu.sync_copy(counts_sh, allc_v)
    base_vec = jnp.zeros((16,), jnp.int32)
    tot_vec = jnp.zeros((16,), jnp.int32)
    for w2 in range(_NW):
        row = allc_v[w2]
        base_vec = base_vec + jnp.where(jnp.int32(w2) < wid, row, 0)
        tot_vec = tot_vec + row
    baseA1 = _lane(base_vec, 0)
    baseA2 = _lane(base_vec, 2)
    tot_fg = _lane(tot_vec, 0)
    tot_bg = _lane(tot_vec, 2)

    def build_pos(s, base, c_self, shift):
        _, cap, out_off = _STREAMS[s]
        for k in range(cap // 128):
            for jj in range(8):
                jvec = k * 128 + jj * 16 + lane
                gpos = shift + base + jvec
                m = (jvec < c_self) & (gpos < cap)
                poss[s][k, pl.ds(jj * 16, 16)] = \
                    jnp.where(m, gpos + out_off, _DUMP)

    def fire_scatter(s):
        _, cap, _ = _STREAMS[s]
        return [pltpu.async_copy(lsts[s].at[pl.ds(k * 128, 128)],
                                 sidx_sh.at[poss[s].at[k]], sem)
                for k in range(cap // 128)]

    build_pos(0, baseA1, cA1, 0)
    build_pos(2, baseA2, cA2, 0)
    descs = fire_scatter(0) + fire_scatter(2)

    def fill_pass(s, lane_id, total):
        cB = lax.fori_loop(0, _VPC, lambda j, c: scan_one(s, c, j),
                           jnp.int32(0))
        plsc.subcore_barrier()
        cnt_v[...] = jnp.where(lane == lane_id, cB, 0)
        pltpu.sync_copy(cnt_v, counts_sh.at[wid])
        plsc.subcore_barrier()
        pltpu.sync_copy(counts_sh, allc_v)
        bv = jnp.zeros((16,), jnp.int32)
        for w2 in range(_NW):
            bv = bv + jnp.where(jnp.int32(w2) < wid, allc_v[w2], 0)
        build_pos(s, _lane(bv, lane_id), cB, total)
        for d in fire_scatter(s):
            d.wait()

    @pl.when(tot_fg < _NUM_FG)
    def _():
        fill_pass(1, 1, tot_fg)

    @pl.when(tot_bg < _NUM_BG)
    def _():
        fill_pass(3, 3, tot_bg)

    for d in descs:
        d.wait()
    plsc.subcore_barrier()

    nslot = (_NUM_FG + _NUM_BG) // _NW
    pltpu.sync_copy(sidx_sh.at[pl.ds(wid * nslot, nslot)], sidx_v)
    sps = (sp0_v, sp1_v, sp2_v, sp3_v)
    sgs = (sg0_v, sg1_v, sg2_v, sg3_v)
    descs = [pltpu.async_copy(vals_hbm.at[sidx_v], sv_v, sem),
             pltpu.async_copy(idxs_hbm.at[sidx_v], mi_v, sem)]
    for c, t in enumerate((pb0_hbm, pb1_hbm, pb2_hbm, pb3_hbm)):
        descs.append(pltpu.async_copy(t.at[sidx_v], sps[c], sem))
    for d in descs:
        d.wait()
    descs = [pltpu.async_copy(gtc_hbm.at[mi_v], sc_v, sem)]
    for c, t in enumerate((gt0_hbm, gt1_hbm, gt2_hbm, gt3_hbm)):
        descs.append(pltpu.async_copy(t.at[mi_v], sgs[c], sem))
    for d in descs:
        d.wait()

    for k in range(nslot // 16):
        rvec = k * 16 + lane
        for c in range(4):
            csplat = jnp.broadcast_to(jnp.int32(c), (16,))
            plsc.store_scatter(outbuf, [rvec, csplat], sps[c][pl.ds(k * 16, 16)])
            plsc.store_scatter(outbuf, [rvec, csplat + 4], sgs[c][pl.ds(k * 16, 16)])
        svv = sv_v[pl.ds(k * 16, 16)]
        plsc.store_scatter(outbuf, [rvec, jnp.broadcast_to(jnp.int32(8), (16,))], svv)
        gtcv = sc_v[pl.ds(k * 16, 16)]
        cls_v[pl.ds(k * 16, 16)] = jnp.where(svv >= 0.5, gtcv, _NUM_CLASSES)

    pltpu.sync_copy(outbuf, out_hbm.at[pl.ds(wid * nslot, nslot)])
    pltpu.sync_copy(cls_v, cls_hbm.at[pl.ds(wid * nslot, nslot)])


def _sc_call():
    mesh = plsc.VectorSubcoreMesh(core_axis_name="c", subcore_axis_name="s",
                                  num_cores=1)
    nslot = (_NUM_FG + _NUM_BG) // _NW
    return pl.kernel(
        _sc_body,
        out_type=[
            jax.ShapeDtypeStruct((_NUM_FG + _NUM_BG, 9), jnp.float32),
            jax.ShapeDtypeStruct((_NUM_FG + _NUM_BG,), jnp.int32),
        ],
        mesh=mesh,
        compiler_params=pltpu.CompilerParams(needs_layout_passes=False,
                                             use_tc_tiling_on_sc=False),
        scratch_types=[
            pltpu.VMEM((_PP,), jnp.int32),
            pltpu.VMEM((_CHUNK,), jnp.int32),
            pltpu.VMEM((_CHUNK,), jnp.int32),
            pltpu.VMEM((_CHUNK,), jnp.int32),
            pltpu.VMEM((_CHUNK,), jnp.int32),
            pltpu.VMEM((_NUM_FG + 16,), jnp.int32),
            pltpu.VMEM((_NUM_FG + 16,), jnp.int32),
            pltpu.VMEM((_NUM_BG + 16,), jnp.int32),
            pltpu.VMEM((_NUM_BG + 16,), jnp.int32),
            pltpu.VMEM((1, 128), jnp.int32),
            pltpu.VMEM((1, 128), jnp.int32),
            pltpu.VMEM((3, 128), jnp.int32),
            pltpu.VMEM((3, 128), jnp.int32),
            pltpu.VMEM((16,), jnp.int32),
            pltpu.VMEM((_NW, 16), jnp.int32),
            pltpu.VMEM((nslot,), jnp.int32),
            pltpu.VMEM((nslot,), jnp.float32),
            pltpu.VMEM((nslot,), jnp.float32),
            pltpu.VMEM((nslot,), jnp.float32),
            pltpu.VMEM((nslot,), jnp.float32),
            pltpu.VMEM((nslot,), jnp.float32),
            pltpu.VMEM((nslot,), jnp.int32),
            pltpu.VMEM((nslot,), jnp.float32),
            pltpu.VMEM((nslot,), jnp.float32),
            pltpu.VMEM((nslot,), jnp.float32),
            pltpu.VMEM((nslot,), jnp.float32),
            pltpu.VMEM((nslot,), jnp.int32),
            pltpu.VMEM((nslot, 9), jnp.float32),
            pltpu.VMEM((nslot,), jnp.int32),
            pltpu.VMEM_SHARED((_NW, 16), jnp.int32),
            pltpu.VMEM_SHARED((_DUMP + 16,), jnp.int32),
            pltpu.SemaphoreType.DMA,
        ],
    )


def kernel(proposal_boxes, gt_boxes, gt_classes):
    perms = [jnp.asarray(p) for p in _perm_consts()]
    pb_pad = jnp.concatenate(
        [proposal_boxes, jnp.zeros((_PP - _P, 4), jnp.float32)], axis=0)
    pbt4 = pb_pad.T
    pbt = pbt4.reshape(4, _ROWS, 128)
    vals, idxs, lab = _iou_call(pbt, gt_boxes)
    gtt = gt_boxes.T
    out, cls = _sc_call()(
        vals.reshape(_PP), idxs.reshape(_PP), lab.reshape(_PP),
        perms[0], perms[1], perms[2], perms[3],
        pbt4[0], pbt4[1], pbt4[2], pbt4[3],
        gtt[0], gtt[1], gtt[2], gtt[3], gt_classes.astype(jnp.int32))
    return out, cls


# speedup vs baseline: 1.4682x; 1.1855x over previous
import functools

import numpy as np
import jax
import jax.numpy as jnp
from jax import lax
from jax.experimental import pallas as pl
from jax.experimental.pallas import tpu as pltpu
from jax.experimental.pallas import tpu_sc as plsc

_NUM_CLASSES = 80
_NUM_FG = 128
_NUM_BG = 384
_P = 20000
_G = 64
_NW = 16
_PP = 20480
_CHUNK = _PP // _NW
_VPC = _CHUNK // 16
_ROWS = _PP // 128
_TC_TILE = 32
_DUMP = _NUM_FG + _NUM_BG


def _rotl(x, r):
    return ((x << np.uint32(r)) | (x >> np.uint32(32 - r))) & np.uint32(0xFFFFFFFF)


def _threefry2x32(key0, key1, x0, x1):
    ks0 = np.uint32(key0)
    ks1 = np.uint32(key1)
    ks2 = ks0 ^ ks1 ^ np.uint32(0x1BD11BDA)
    x0 = (x0 + ks0).astype(np.uint32)
    x1 = (x1 + ks1).astype(np.uint32)
    rots = ([13, 15, 26, 6], [17, 29, 16, 24])
    inj = [(ks1, ks2), (ks2, ks0), (ks0, ks1), (ks1, ks2), (ks2, ks0)]
    for i in range(5):
        for r in rots[i % 2]:
            x0 = (x0 + x1).astype(np.uint32)
            x1 = _rotl(x1, r)
            x1 = x1 ^ x0
        a, b = inj[i]
        x0 = (x0 + a).astype(np.uint32)
        x1 = (x1 + b + np.uint32(i + 1)).astype(np.uint32)
    return x0, x1


def _fry_uniform(key, n):
    o0, o1 = _threefry2x32(key[0], key[1],
                           np.zeros(n, np.uint32), np.arange(n, dtype=np.uint32))
    bits = o0 ^ o1
    return ((bits >> np.uint32(9)) | np.uint32(0x3F800000)).view(np.float32) \
        - np.float32(1.0)


@functools.cache
def _perm_consts():
    o0, o1 = _threefry2x32(np.uint32(0), np.uint32(1),
                           np.zeros(2, np.uint32), np.arange(2, dtype=np.uint32))
    u1 = _fry_uniform((o0[0], o1[0]), _P)
    u2 = _fry_uniform((o0[1], o1[1]), _P)
    keys = [
        (np.float32(1.0) + u1),
        u1,
        (np.float32(1.0) + u2),
        u2,
    ]
    pad = np.arange(_P, _PP, dtype=np.int32)
    return [np.concatenate([np.argsort(-k, kind="stable").astype(np.int32), pad])
            for k in keys]


def _iou_body(pbt_ref, gt_ref, vals_ref, idxs_ref, lab_ref):
    x0 = pbt_ref[0]
    y0 = pbt_ref[1]
    x1 = pbt_ref[2]
    y1 = pbt_ref[3]
    area2 = (x1 - x0) * (y1 - y0)

    vals = jnp.zeros(x0.shape, jnp.float32)
    idxs = jnp.zeros(x0.shape, jnp.int32)
    for g in range(_G):
        gx0 = gt_ref[g, 0]
        gy0 = gt_ref[g, 1]
        gx1 = gt_ref[g, 2]
        gy1 = gt_ref[g, 3]
        a1 = (gx1 - gx0) * (gy1 - gy0)
        w = jnp.maximum(jnp.minimum(gx1, x1) - jnp.maximum(gx0, x0), 0.0)
        h = jnp.maximum(jnp.minimum(gy1, y1) - jnp.maximum(gy0, y0), 0.0)
        inter = w * h
        iou = inter / ((a1 + area2) - inter)
        better = iou > vals
        vals = jnp.where(better, iou, vals)
        idxs = jnp.where(better, g, idxs)
    vals_ref[...] = vals
    idxs_ref[...] = idxs
    r = lax.broadcasted_iota(jnp.int32, x0.shape, 0)
    c = lax.broadcasted_iota(jnp.int32, x0.shape, 1)
    gidx = (pl.program_id(0) * _TC_TILE + r) * 128 + c
    fg = (vals >= 0.5).astype(jnp.int32)
    lab_ref[...] = jnp.where(gidx < _P, fg, -1)


def _iou_call(pbt, gt_boxes):
    n_steps = _ROWS // _TC_TILE
    return pl.pallas_call(
        _iou_body,
        grid=(n_steps,),
        in_specs=[
            pl.BlockSpec((4, _TC_TILE, 128), lambda i: (0, i, 0)),
            pl.BlockSpec(memory_space=pltpu.SMEM),
        ],
        out_specs=[
            pl.BlockSpec((_TC_TILE, 128), lambda i: (i, 0)),
            pl.BlockSpec((_TC_TILE, 128), lambda i: (i, 0)),
            pl.BlockSpec((_TC_TILE, 128), lambda i: (i, 0)),
        ],
        out_shape=[
            jax.ShapeDtypeStruct((_ROWS, 128), jnp.float32),
            jax.ShapeDtypeStruct((_ROWS, 128), jnp.int32),
            jax.ShapeDtypeStruct((_ROWS, 128), jnp.int32),
        ],
    )(pbt, gt_boxes)


_STREAMS = (
    (1, _NUM_FG, 0),
    (0, _NUM_FG, 0),
    (0, _NUM_BG, _NUM_FG),
    (1, _NUM_BG, _NUM_FG),
)


def _lane(v, s):
    return jnp.sum(jnp.where(lax.iota(jnp.int32, 16) == s, v, 0))


def _first(v):
    return lax.squeeze(lax.slice(v, (0,), (1,)), (0,))


def _sc_body(vals_hbm, idxs_hbm, lab_hbm, p0_hbm, p1_hbm, p2_hbm, p3_hbm,
             pb0_hbm, pb1_hbm, pb2_hbm, pb3_hbm,
             gt0_hbm, gt1_hbm, gt2_hbm, gt3_hbm, gtc_hbm,
             out_hbm, cls_hbm,
             label_v, perm0_v, perm1_v, perm2_v, perm3_v,
             lst0, lst1, lst2, lst3,
             pos0, pos1, pos2, pos3,
             cnt_v, allc_v,
             sidx_v, sp0_v, sp1_v, sp2_v, sp3_v, sv_v, mi_v,
             sg0_v, sg1_v, sg2_v, sg3_v, sc_v, outbuf, cls_v,
             counts_sh, sidx_sh, sem):
    wid = lax.axis_index("s")
    lane = lax.iota(jnp.int32, 16)

    perm_hbms = (p0_hbm, p1_hbm, p2_hbm, p3_hbm)
    perm_vs = (perm0_v, perm1_v, perm2_v, perm3_v)
    lsts = (lst0, lst1, lst2, lst3)
    poss = (pos0, pos1, pos2, pos3)
    descs = [pltpu.async_copy(lab_hbm, label_v, sem)]
    for s in range(4):
        descs.append(pltpu.async_copy(
            perm_hbms[s].at[pl.ds(wid * _CHUNK, _CHUNK)], perm_vs[s], sem))
    for d in descs:
        d.wait()

    def scan_one(s, cnt, j):
        target, cap, _ = _STREAMS[s]
        idxv = perm_vs[s][pl.ds(j * 16, 16)]
        labv = plsc.load_gather(label_v, [idxv])
        m = labv == target
        off = jnp.minimum(cnt, cap)
        plsc.store_compressed(lsts[s].at[pl.ds(off, 16)], idxv, mask=m)
        return cnt + _first(plsc.all_reduce_population_count(m))

    def cbody(j, cnts):
        cA1, cA2 = cnts
        for u in range(2):
            cA1 = scan_one(0, cA1, j * 2 + u)
            cA2 = scan_one(2, cA2, j * 2 + u)
        return cA1, cA2

    cA1, cA2 = lax.fori_loop(0, _VPC // 2, cbody, (jnp.int32(0),) * 2)

    cnt_v[...] = jnp.where(lane == 0, cA1, 0) + jnp.where(lane == 2, cA2, 0)
    pltpu.sync_copy(cnt_v, counts_sh.at[wid])
    plsc.subcore_barrier()
    pltpu.sync_copy(counts_sh, allc_v)
    base_vec = jnp.zeros((16,), jnp.int32)
    tot_vec = jnp.zeros((16,), jnp.int32)
    for w2 in range(_NW):
        row = allc_v[w2]
        base_vec = base_vec + jnp.where(jnp.int32(w2) < wid, row, 0)
        tot_vec = tot_vec + row
    baseA1 = _lane(base_vec, 0)
    baseA2 = _lane(base_vec, 2)
    tot_fg = _lane(tot_vec, 0)
    tot_bg = _lane(tot_vec, 2)

    def build_pos(s, base, c_self, shift):
        _, cap, out_off = _STREAMS[s]
        for k in range(cap // 128):
            for jj in range(8):
                jvec = k * 128 + jj * 16 + lane
                gpos = shift + base + jvec
                m = (jvec < c_self) & (gpos < cap)
                poss[s][k, pl.ds(jj * 16, 16)] = \
                    jnp.where(m, gpos + out_off, _DUMP)

    def fire_scatter(s):
        _, cap, _ = _STREAMS[s]
        return [pltpu.async_copy(lsts[s].at[pl.ds(k * 128, 128)],
                                 sidx_sh.at[poss[s].at[k]], sem)
                for k in range(cap // 128)]

    build_pos(0, baseA1, cA1, 0)
    build_pos(2, baseA2, cA2, 0)
    descs = fire_scatter(0) + fire_scatter(2)

    def fill_pass(s, lane_id, total):
        cB = lax.fori_loop(0, _VPC, lambda j, c: scan_one(s, c, j),
                           jnp.int32(0))
        plsc.subcore_barrier()
        cnt_v[...] = jnp.where(lane == lane_id, cB, 0)
        pltpu.sync_copy(cnt_v, counts_sh.at[wid])
        plsc.subcore_barrier()
        pltpu.sync_copy(counts_sh, allc_v)
        bv = jnp.zeros((16,), jnp.int32)
        for w2 in range(_NW):
            bv = bv + jnp.where(jnp.int32(w2) < wid, allc_v[w2], 0)
        build_pos(s, _lane(bv, lane_id), cB, total)
        for d in fire_scatter(s):
            d.wait()

    @pl.when(tot_fg < _NUM_FG)
    def _():
        fill_pass(1, 1, tot_fg)

    @pl.when(tot_bg < _NUM_BG)
    def _():
        fill_pass(3, 3, tot_bg)

    for d in descs:
        d.wait()
    plsc.subcore_barrier()

    nslot = (_NUM_FG + _NUM_BG) // _NW
    pltpu.sync_copy(sidx_sh.at[pl.ds(wid * nslot, nslot)], sidx_v)
    sps = (sp0_v, sp1_v, sp2_v, sp3_v)
    sgs = (sg0_v, sg1_v, sg2_v, sg3_v)
    descs = [pltpu.async_copy(vals_hbm.at[sidx_v], sv_v, sem),
             pltpu.async_copy(idxs_hbm.at[sidx_v], mi_v, sem)]
    for c, t in enumerate((pb0_hbm, pb1_hbm, pb2_hbm, pb3_hbm)):
        descs.append(pltpu.async_copy(t.at[sidx_v], sps[c], sem))
    for d in descs:
        d.wait()
    descs = [pltpu.async_copy(gtc_hbm.at[mi_v], sc_v, sem)]
    for c, t in enumerate((gt0_hbm, gt1_hbm, gt2_hbm, gt3_hbm)):
        descs.append(pltpu.async_copy(t.at[mi_v], sgs[c], sem))
    for d in descs:
        d.wait()

    for k in range(nslot // 16):
        rvec = k * 16 + lane
        for c in range(4):
            csplat = jnp.broadcast_to(jnp.int32(c), (16,))
            plsc.store_scatter(outbuf, [rvec, csplat], sps[c][pl.ds(k * 16, 16)])
            plsc.store_scatter(outbuf, [rvec, csplat + 4], sgs[c][pl.ds(k * 16, 16)])
        svv = sv_v[pl.ds(k * 16, 16)]
        plsc.store_scatter(outbuf, [rvec, jnp.broadcast_to(jnp.int32(8), (16,))], svv)
        gtcv = sc_v[pl.ds(k * 16, 16)]
        cls_v[pl.ds(k * 16, 16)] = jnp.where(svv >= 0.5, gtcv, _NUM_CLASSES)

    pltpu.sync_copy(outbuf, out_hbm.at[pl.ds(wid * nslot, nslot)])
    pltpu.sync_copy(cls_v, cls_hbm.at[pl.ds(wid * nslot, nslot)])


def _sc_call():
    mesh = plsc.VectorSubcoreMesh(core_axis_name="c", subcore_axis_name="s",
                                  num_cores=1)
    nslot = (_NUM_FG + _NUM_BG) // _NW
    return pl.kernel(
        _sc_body,
        out_type=[
            jax.ShapeDtypeStruct((_NUM_FG + _NUM_BG, 9), jnp.float32),
            jax.ShapeDtypeStruct((_NUM_FG + _NUM_BG,), jnp.int32),
        ],
        mesh=mesh,
        compiler_params=pltpu.CompilerParams(needs_layout_passes=False,
                                             use_tc_tiling_on_sc=False),
        scratch_types=[
            pltpu.VMEM((_PP,), jnp.int32),
            pltpu.VMEM((_CHUNK,), jnp.int32),
            pltpu.VMEM((_CHUNK,), jnp.int32),
            pltpu.VMEM((_CHUNK,), jnp.int32),
            pltpu.VMEM((_CHUNK,), jnp.int32),
            pltpu.VMEM((_NUM_FG + 16,), jnp.int32),
            pltpu.VMEM((_NUM_FG + 16,), jnp.int32),
            pltpu.VMEM((_NUM_BG + 16,), jnp.int32),
            pltpu.VMEM((_NUM_BG + 16,), jnp.int32),
            pltpu.VMEM((1, 128), jnp.int32),
            pltpu.VMEM((1, 128), jnp.int32),
            pltpu.VMEM((3, 128), jnp.int32),
            pltpu.VMEM((3, 128), jnp.int32),
            pltpu.VMEM((16,), jnp.int32),
            pltpu.VMEM((_NW, 16), jnp.int32),
            pltpu.VMEM((nslot,), jnp.int32),
            pltpu.VMEM((nslot,), jnp.float32),
            pltpu.VMEM((nslot,), jnp.float32),
            pltpu.VMEM((nslot,), jnp.float32),
            pltpu.VMEM((nslot,), jnp.float32),
            pltpu.VMEM((nslot,), jnp.float32),
            pltpu.VMEM((nslot,), jnp.int32),
            pltpu.VMEM((nslot,), jnp.float32),
            pltpu.VMEM((nslot,), jnp.float32),
            pltpu.VMEM((nslot,), jnp.float32),
            pltpu.VMEM((nslot,), jnp.float32),
            pltpu.VMEM((nslot,), jnp.int32),
            pltpu.VMEM((nslot, 9), jnp.float32),
            pltpu.VMEM((nslot,), jnp.int32),
            pltpu.VMEM_SHARED((_NW, 16), jnp.int32),
            pltpu.VMEM_SHARED((_DUMP + 16,), jnp.int32),
            pltpu.SemaphoreType.DMA,
        ],
    )


def kernel(proposal_boxes, gt_boxes, gt_classes):
    perms = [jnp.asarray(p) for p in _perm_consts()]
    pb_pad = jnp.concatenate(
        [proposal_boxes, jnp.zeros((_PP - _P, 4), jnp.float32)], axis=0)
    pbt4 = pb_pad.T
    pbt = pbt4.reshape(4, _ROWS, 128)
    vals, idxs, lab = _iou_call(pbt, gt_boxes)
    gtt = gt_boxes.T
    out, cls = _sc_call()(
        vals.reshape(_PP), idxs.reshape(_PP), lab.reshape(_PP),
        perms[0], perms[1], perms[2], perms[3],
        pbt4[0], pbt4[1], pbt4[2], pbt4[3],
        gtt[0], gtt[1], gtt[2], gtt[3], gt_classes.astype(jnp.int32))
    return out, cls


# speedup vs baseline: 1.4884x; 1.0137x over previous
import functools

import numpy as np
import jax
import jax.numpy as jnp
from jax import lax
from jax.experimental import pallas as pl
from jax.experimental.pallas import tpu as pltpu
from jax.experimental.pallas import tpu_sc as plsc

_NUM_CLASSES = 80
_NUM_FG = 128
_NUM_BG = 384
_P = 20000
_G = 64
_NW = 16
_PP = 20480
_CHUNK = _PP // _NW
_VPC = _CHUNK // 16
_ROWS = _PP // 128
_TC_TILE = 32
_DUMP = _NUM_FG + _NUM_BG


def _rotl(x, r):
    return ((x << np.uint32(r)) | (x >> np.uint32(32 - r))) & np.uint32(0xFFFFFFFF)


def _threefry2x32(key0, key1, x0, x1):
    ks0 = np.uint32(key0)
    ks1 = np.uint32(key1)
    ks2 = ks0 ^ ks1 ^ np.uint32(0x1BD11BDA)
    x0 = (x0 + ks0).astype(np.uint32)
    x1 = (x1 + ks1).astype(np.uint32)
    rots = ([13, 15, 26, 6], [17, 29, 16, 24])
    inj = [(ks1, ks2), (ks2, ks0), (ks0, ks1), (ks1, ks2), (ks2, ks0)]
    for i in range(5):
        for r in rots[i % 2]:
            x0 = (x0 + x1).astype(np.uint32)
            x1 = _rotl(x1, r)
            x1 = x1 ^ x0
        a, b = inj[i]
        x0 = (x0 + a).astype(np.uint32)
        x1 = (x1 + b + np.uint32(i + 1)).astype(np.uint32)
    return x0, x1


def _fry_uniform(key, n):
    o0, o1 = _threefry2x32(key[0], key[1],
                           np.zeros(n, np.uint32), np.arange(n, dtype=np.uint32))
    bits = o0 ^ o1
    return ((bits >> np.uint32(9)) | np.uint32(0x3F800000)).view(np.float32) \
        - np.float32(1.0)


@functools.cache
def _perm_consts():
    o0, o1 = _threefry2x32(np.uint32(0), np.uint32(1),
                           np.zeros(2, np.uint32), np.arange(2, dtype=np.uint32))
    u1 = _fry_uniform((o0[0], o1[0]), _P)
    u2 = _fry_uniform((o0[1], o1[1]), _P)
    keys = [
        (np.float32(1.0) + u1),
        u1,
        (np.float32(1.0) + u2),
        u2,
    ]
    pad = np.arange(_P, _PP, dtype=np.int32)
    return [np.concatenate([np.argsort(-k, kind="stable").astype(np.int32), pad])
            for k in keys]


def _iou_body(pbt_ref, gt_ref, vals_ref, idxs_ref, lab_ref):
    x0 = pbt_ref[0]
    y0 = pbt_ref[1]
    x1 = pbt_ref[2]
    y1 = pbt_ref[3]
    area2 = (x1 - x0) * (y1 - y0)

    vals = jnp.zeros(x0.shape, jnp.float32)
    idxs = jnp.zeros(x0.shape, jnp.int32)
    for g in range(_G):
        gx0 = gt_ref[g, 0]
        gy0 = gt_ref[g, 1]
        gx1 = gt_ref[g, 2]
        gy1 = gt_ref[g, 3]
        a1 = (gx1 - gx0) * (gy1 - gy0)
        w = jnp.maximum(jnp.minimum(gx1, x1) - jnp.maximum(gx0, x0), 0.0)
        h = jnp.maximum(jnp.minimum(gy1, y1) - jnp.maximum(gy0, y0), 0.0)
        inter = w * h
        iou = inter / ((a1 + area2) - inter)
        better = iou > vals
        vals = jnp.where(better, iou, vals)
        idxs = jnp.where(better, g, idxs)
    vals_ref[...] = vals
    idxs_ref[...] = idxs
    r = lax.broadcasted_iota(jnp.int32, x0.shape, 0)
    c = lax.broadcasted_iota(jnp.int32, x0.shape, 1)
    gidx = (pl.program_id(0) * _TC_TILE + r) * 128 + c
    fg = (vals >= 0.5).astype(jnp.int32)
    lab_ref[...] = jnp.where(gidx < _P, fg, -1)


def _iou_call(pbt, gt_boxes):
    n_steps = _ROWS // _TC_TILE
    return pl.pallas_call(
        _iou_body,
        grid=(n_steps,),
        in_specs=[
            pl.BlockSpec((4, _TC_TILE, 128), lambda i: (0, i, 0)),
            pl.BlockSpec(memory_space=pltpu.SMEM),
        ],
        out_specs=[
            pl.BlockSpec((_TC_TILE, 128), lambda i: (i, 0)),
            pl.BlockSpec((_TC_TILE, 128), lambda i: (i, 0)),
            pl.BlockSpec((_TC_TILE, 128), lambda i: (i, 0)),
        ],
        out_shape=[
            jax.ShapeDtypeStruct((_ROWS, 128), jnp.float32),
            jax.ShapeDtypeStruct((_ROWS, 128), jnp.int32),
            jax.ShapeDtypeStruct((_ROWS, 128), jnp.int32),
        ],
    )(pbt, gt_boxes)


_STREAMS = (
    (1, _NUM_FG, 0),
    (0, _NUM_FG, 0),
    (0, _NUM_BG, _NUM_FG),
    (1, _NUM_BG, _NUM_FG),
)


def _lane(v, s):
    return jnp.sum(jnp.where(lax.iota(jnp.int32, 16) == s, v, 0))


def _first(v):
    return lax.squeeze(lax.slice(v, (0,), (1,)), (0,))


def _sc_body(vals_hbm, idxs_hbm, lab_hbm, perm_hbm,
             pb0_hbm, pb1_hbm, pb2_hbm, pb3_hbm,
             gt0_hbm, gt1_hbm, gt2_hbm, gt3_hbm, gtc_hbm,
             out_hbm, cls_hbm,
             label_v, perm0_v, perm1_v, perm2_v, perm3_v,
             lst0, lst1, lst2, lst3,
             pos0, pos1, pos2, pos3,
             cnt_v, allc_v,
             sidx_v, sp0_v, sp1_v, sp2_v, sp3_v, sv_v, mi_v,
             sg0_v, sg1_v, sg2_v, sg3_v, sc_v, outbuf, cls_v,
             counts_sh, sidx_sh, sem):
    wid = lax.axis_index("s")
    lane = lax.iota(jnp.int32, 16)

    perm_vs = (perm0_v, perm1_v, perm2_v, perm3_v)
    lsts = (lst0, lst1, lst2, lst3)
    poss = (pos0, pos1, pos2, pos3)
    descs = [pltpu.async_copy(lab_hbm, label_v, sem)]
    for s in range(4):
        descs.append(pltpu.async_copy(
            perm_hbm.at[pl.ds(s * _PP + wid * _CHUNK, _CHUNK)], perm_vs[s], sem))
    for d in descs:
        d.wait()

    def scan_one(s, cnt, j):
        target, cap, _ = _STREAMS[s]
        idxv = perm_vs[s][pl.ds(j * 16, 16)]
        labv = plsc.load_gather(label_v, [idxv])
        m = labv == target
        off = jnp.minimum(cnt, cap)
        plsc.store_compressed(lsts[s].at[pl.ds(off, 16)], idxv, mask=m)
        return cnt + _first(plsc.all_reduce_population_count(m))

    def cbody(j, cnts):
        cA1, cA2 = cnts
        for u in range(2):
            cA1 = scan_one(0, cA1, j * 2 + u)
            cA2 = scan_one(2, cA2, j * 2 + u)
        return cA1, cA2

    cA1, cA2 = lax.fori_loop(0, _VPC // 2, cbody, (jnp.int32(0),) * 2)

    cnt_v[...] = jnp.where(lane == 0, cA1, 0) + jnp.where(lane == 2, cA2, 0)
    pltpu.sync_copy(cnt_v, counts_sh.at[wid])
    plsc.subcore_barrier()
    pltpu.sync_copy(counts_sh, allc_v)
    base_vec = jnp.zeros((16,), jnp.int32)
    tot_vec = jnp.zeros((16,), jnp.int32)
    for w2 in range(_NW):
        row = allc_v[w2]
        base_vec = base_vec + jnp.where(jnp.int32(w2) < wid, row, 0)
        tot_vec = tot_vec + row
    baseA1 = _lane(base_vec, 0)
    baseA2 = _lane(base_vec, 2)
    tot_fg = _lane(tot_vec, 0)
    tot_bg = _lane(tot_vec, 2)

    def build_pos(s, base, c_self, shift):
        _, cap, out_off = _STREAMS[s]
        for k in range(cap // 128):
            for jj in range(8):
                jvec = k * 128 + jj * 16 + lane
                gpos = shift + base + jvec
                m = (jvec < c_self) & (gpos < cap)
                poss[s][k, pl.ds(jj * 16, 16)] = \
                    jnp.where(m, gpos + out_off, _DUMP)

    def fire_scatter(s):
        _, cap, _ = _STREAMS[s]
        return [pltpu.async_copy(lsts[s].at[pl.ds(k * 128, 128)],
                                 sidx_sh.at[poss[s].at[k]], sem)
                for k in range(cap // 128)]

    build_pos(0, baseA1, cA1, 0)
    build_pos(2, baseA2, cA2, 0)
    descs = fire_scatter(0) + fire_scatter(2)

    def fill_pass(s, lane_id, total):
        cB = lax.fori_loop(0, _VPC, lambda j, c: scan_one(s, c, j),
                           jnp.int32(0))
        plsc.subcore_barrier()
        cnt_v[...] = jnp.where(lane == lane_id, cB, 0)
        pltpu.sync_copy(cnt_v, counts_sh.at[wid])
        plsc.subcore_barrier()
        pltpu.sync_copy(counts_sh, allc_v)
        bv = jnp.zeros((16,), jnp.int32)
        for w2 in range(_NW):
            bv = bv + jnp.where(jnp.int32(w2) < wid, allc_v[w2], 0)
        build_pos(s, _lane(bv, lane_id), cB, total)
        for d in fire_scatter(s):
            d.wait()

    @pl.when(tot_fg < _NUM_FG)
    def _():
        fill_pass(1, 1, tot_fg)

    @pl.when(tot_bg < _NUM_BG)
    def _():
        fill_pass(3, 3, tot_bg)

    for d in descs:
        d.wait()
    plsc.subcore_barrier()

    nslot = (_NUM_FG + _NUM_BG) // _NW
    pltpu.sync_copy(sidx_sh.at[pl.ds(wid * nslot, nslot)], sidx_v)
    sps = (sp0_v, sp1_v, sp2_v, sp3_v)
    sgs = (sg0_v, sg1_v, sg2_v, sg3_v)
    descs = [pltpu.async_copy(vals_hbm.at[sidx_v], sv_v, sem),
             pltpu.async_copy(idxs_hbm.at[sidx_v], mi_v, sem)]
    for c, t in enumerate((pb0_hbm, pb1_hbm, pb2_hbm, pb3_hbm)):
        descs.append(pltpu.async_copy(t.at[sidx_v], sps[c], sem))
    for d in descs:
        d.wait()
    descs = [pltpu.async_copy(gtc_hbm.at[mi_v], sc_v, sem)]
    for c, t in enumerate((gt0_hbm, gt1_hbm, gt2_hbm, gt3_hbm)):
        descs.append(pltpu.async_copy(t.at[mi_v], sgs[c], sem))
    for d in descs:
        d.wait()

    for k in range(nslot // 16):
        rvec = k * 16 + lane
        for c in range(4):
            csplat = jnp.broadcast_to(jnp.int32(c), (16,))
            plsc.store_scatter(outbuf, [rvec, csplat], sps[c][pl.ds(k * 16, 16)])
            plsc.store_scatter(outbuf, [rvec, csplat + 4], sgs[c][pl.ds(k * 16, 16)])
        svv = sv_v[pl.ds(k * 16, 16)]
        plsc.store_scatter(outbuf, [rvec, jnp.broadcast_to(jnp.int32(8), (16,))], svv)
        gtcv = sc_v[pl.ds(k * 16, 16)]
        cls_v[pl.ds(k * 16, 16)] = jnp.where(svv >= 0.5, gtcv, _NUM_CLASSES)

    pltpu.sync_copy(outbuf, out_hbm.at[pl.ds(wid * nslot, nslot)])
    pltpu.sync_copy(cls_v, cls_hbm.at[pl.ds(wid * nslot, nslot)])


def _sc_call():
    mesh = plsc.VectorSubcoreMesh(core_axis_name="c", subcore_axis_name="s",
                                  num_cores=1)
    nslot = (_NUM_FG + _NUM_BG) // _NW
    return pl.kernel(
        _sc_body,
        out_type=[
            jax.ShapeDtypeStruct((_NUM_FG + _NUM_BG, 9), jnp.float32),
            jax.ShapeDtypeStruct((_NUM_FG + _NUM_BG,), jnp.int32),
        ],
        mesh=mesh,
        compiler_params=pltpu.CompilerParams(needs_layout_passes=False,
                                             use_tc_tiling_on_sc=False),
        scratch_types=[
            pltpu.VMEM((_PP,), jnp.int32),
            pltpu.VMEM((_CHUNK,), jnp.int32),
            pltpu.VMEM((_CHUNK,), jnp.int32),
            pltpu.VMEM((_CHUNK,), jnp.int32),
            pltpu.VMEM((_CHUNK,), jnp.int32),
            pltpu.VMEM((_NUM_FG + 16,), jnp.int32),
            pltpu.VMEM((_NUM_FG + 16,), jnp.int32),
            pltpu.VMEM((_NUM_BG + 16,), jnp.int32),
            pltpu.VMEM((_NUM_BG + 16,), jnp.int32),
            pltpu.VMEM((1, 128), jnp.int32),
            pltpu.VMEM((1, 128), jnp.int32),
            pltpu.VMEM((3, 128), jnp.int32),
            pltpu.VMEM((3, 128), jnp.int32),
            pltpu.VMEM((16,), jnp.int32),
            pltpu.VMEM((_NW, 16), jnp.int32),
            pltpu.VMEM((nslot,), jnp.int32),
            pltpu.VMEM((nslot,), jnp.float32),
            pltpu.VMEM((nslot,), jnp.float32),
            pltpu.VMEM((nslot,), jnp.float32),
            pltpu.VMEM((nslot,), jnp.float32),
            pltpu.VMEM((nslot,), jnp.float32),
            pltpu.VMEM((nslot,), jnp.int32),
            pltpu.VMEM((nslot,), jnp.float32),
            pltpu.VMEM((nslot,), jnp.float32),
            pltpu.VMEM((nslot,), jnp.float32),
            pltpu.VMEM((nslot,), jnp.float32),
            pltpu.VMEM((nslot,), jnp.int32),
            pltpu.VMEM((nslot, 9), jnp.float32),
            pltpu.VMEM((nslot,), jnp.int32),
            pltpu.VMEM_SHARED((_NW, 16), jnp.int32),
            pltpu.VMEM_SHARED((_DUMP + 16,), jnp.int32),
            pltpu.SemaphoreType.DMA,
        ],
    )


def kernel(proposal_boxes, gt_boxes, gt_classes):
    perms = [jnp.asarray(p) for p in _perm_consts()]
    pb_pad = jnp.concatenate(
        [proposal_boxes, jnp.zeros((_PP - _P, 4), jnp.float32)], axis=0)
    pbt4 = pb_pad.T
    pbt = pbt4.reshape(4, _ROWS, 128)
    vals, idxs, lab = _iou_call(pbt, gt_boxes)
    gtt = gt_boxes.T
    out, cls = _sc_call()(
        vals.reshape(_PP), idxs.reshape(_PP), lab.reshape(_PP),
        jnp.concatenate(perms),
        pbt4[0], pbt4[1], pbt4[2], pbt4[3],
        gtt[0], gtt[1], gtt[2], gtt[3], gt_classes.astype(jnp.int32))
    return out, cls


# speedup vs baseline: 1.5089x; 1.0138x over previous
import functools

import numpy as np
import jax
import jax.numpy as jnp
from jax import lax
from jax.experimental import pallas as pl
from jax.experimental.pallas import tpu as pltpu
from jax.experimental.pallas import tpu_sc as plsc

_NUM_CLASSES = 80
_NUM_FG = 128
_NUM_BG = 384
_P = 20000
_G = 64
_NW = 16
_PP = 20480
_CHUNK = _PP // _NW
_VPC = _CHUNK // 16
_ROWS = _PP // 128
_TC_TILE = 40
_DUMP = _NUM_FG + _NUM_BG


def _rotl(x, r):
    return ((x << np.uint32(r)) | (x >> np.uint32(32 - r))) & np.uint32(0xFFFFFFFF)


def _threefry2x32(key0, key1, x0, x1):
    ks0 = np.uint32(key0)
    ks1 = np.uint32(key1)
    ks2 = ks0 ^ ks1 ^ np.uint32(0x1BD11BDA)
    x0 = (x0 + ks0).astype(np.uint32)
    x1 = (x1 + ks1).astype(np.uint32)
    rots = ([13, 15, 26, 6], [17, 29, 16, 24])
    inj = [(ks1, ks2), (ks2, ks0), (ks0, ks1), (ks1, ks2), (ks2, ks0)]
    for i in range(5):
        for r in rots[i % 2]:
            x0 = (x0 + x1).astype(np.uint32)
            x1 = _rotl(x1, r)
            x1 = x1 ^ x0
        a, b = inj[i]
        x0 = (x0 + a).astype(np.uint32)
        x1 = (x1 + b + np.uint32(i + 1)).astype(np.uint32)
    return x0, x1


def _fry_uniform(key, n):
    o0, o1 = _threefry2x32(key[0], key[1],
                           np.zeros(n, np.uint32), np.arange(n, dtype=np.uint32))
    bits = o0 ^ o1
    return ((bits >> np.uint32(9)) | np.uint32(0x3F800000)).view(np.float32) \
        - np.float32(1.0)


@functools.cache
def _perm_consts():
    o0, o1 = _threefry2x32(np.uint32(0), np.uint32(1),
                           np.zeros(2, np.uint32), np.arange(2, dtype=np.uint32))
    u1 = _fry_uniform((o0[0], o1[0]), _P)
    u2 = _fry_uniform((o0[1], o1[1]), _P)
    keys = [
        (np.float32(1.0) + u1),
        u1,
        (np.float32(1.0) + u2),
        u2,
    ]
    pad = np.arange(_P, _PP, dtype=np.int32)
    return [np.concatenate([np.argsort(-k, kind="stable").astype(np.int32), pad])
            for k in keys]


def _iou_body(pbt_ref, gt_ref, vals_ref, idxs_ref, lab_ref):
    x0 = pbt_ref[0]
    y0 = pbt_ref[1]
    x1 = pbt_ref[2]
    y1 = pbt_ref[3]
    area2 = (x1 - x0) * (y1 - y0)

    vals = jnp.zeros(x0.shape, jnp.float32)
    idxs = jnp.zeros(x0.shape, jnp.int32)
    for g in range(_G):
        gx0 = gt_ref[g, 0]
        gy0 = gt_ref[g, 1]
        gx1 = gt_ref[g, 2]
        gy1 = gt_ref[g, 3]
        a1 = (gx1 - gx0) * (gy1 - gy0)
        w = jnp.maximum(jnp.minimum(gx1, x1) - jnp.maximum(gx0, x0), 0.0)
        h = jnp.maximum(jnp.minimum(gy1, y1) - jnp.maximum(gy0, y0), 0.0)
        inter = w * h
        iou = inter / ((a1 + area2) - inter)
        better = iou > vals
        vals = jnp.where(better, iou, vals)
        idxs = jnp.where(better, g, idxs)
    vals_ref[...] = vals
    idxs_ref[...] = idxs
    r = lax.broadcasted_iota(jnp.int32, x0.shape, 0)
    c = lax.broadcasted_iota(jnp.int32, x0.shape, 1)
    gidx = (pl.program_id(0) * _TC_TILE + r) * 128 + c
    fg = (vals >= 0.5).astype(jnp.int32)
    lab_ref[...] = jnp.where(gidx < _P, fg, -1)


def _iou_call(pbt, gt_boxes):
    n_steps = _ROWS // _TC_TILE
    return pl.pallas_call(
        _iou_body,
        grid=(n_steps,),
        in_specs=[
            pl.BlockSpec((4, _TC_TILE, 128), lambda i: (0, i, 0)),
            pl.BlockSpec(memory_space=pltpu.SMEM),
        ],
        out_specs=[
            pl.BlockSpec((_TC_TILE, 128), lambda i: (i, 0)),
            pl.BlockSpec((_TC_TILE, 128), lambda i: (i, 0)),
            pl.BlockSpec((_TC_TILE, 128), lambda i: (i, 0)),
        ],
        out_shape=[
            jax.ShapeDtypeStruct((_ROWS, 128), jnp.float32),
            jax.ShapeDtypeStruct((_ROWS, 128), jnp.int32),
            jax.ShapeDtypeStruct((_ROWS, 128), jnp.int32),
        ],
    )(pbt, gt_boxes)


_STREAMS = (
    (1, _NUM_FG, 0),
    (0, _NUM_FG, 0),
    (0, _NUM_BG, _NUM_FG),
    (1, _NUM_BG, _NUM_FG),
)


def _lane(v, s):
    return jnp.sum(jnp.where(lax.iota(jnp.int32, 16) == s, v, 0))


def _first(v):
    return lax.squeeze(lax.slice(v, (0,), (1,)), (0,))


def _sc_body(vals_hbm, idxs_hbm, lab_hbm, perm_hbm,
             pb0_hbm, pb1_hbm, pb2_hbm, pb3_hbm,
             gt0_hbm, gt1_hbm, gt2_hbm, gt3_hbm, gtc_hbm,
             out_hbm, cls_hbm,
             label_v, perm0_v, perm1_v, perm2_v, perm3_v,
             lst0, lst1, lst2, lst3,
             pos0, pos1, pos2, pos3,
             cnt_v, allc_v,
             sidx_v, sp0_v, sp1_v, sp2_v, sp3_v, sv_v, mi_v,
             sg0_v, sg1_v, sg2_v, sg3_v, sc_v, outbuf, cls_v,
             counts_sh, sidx_sh, sem):
    wid = lax.axis_index("s")
    lane = lax.iota(jnp.int32, 16)

    perm_vs = (perm0_v, perm1_v, perm2_v, perm3_v)
    lsts = (lst0, lst1, lst2, lst3)
    poss = (pos0, pos1, pos2, pos3)
    descs = [pltpu.async_copy(lab_hbm, label_v, sem)]
    for s in range(4):
        descs.append(pltpu.async_copy(
            perm_hbm.at[pl.ds(s * _PP + wid * _CHUNK, _CHUNK)], perm_vs[s], sem))
    for d in descs:
        d.wait()

    def scan_one(s, cnt, j):
        target, cap, _ = _STREAMS[s]
        idxv = perm_vs[s][pl.ds(j * 16, 16)]
        labv = plsc.load_gather(label_v, [idxv])
        m = labv == target
        off = jnp.minimum(cnt, cap)
        plsc.store_compressed(lsts[s].at[pl.ds(off, 16)], idxv, mask=m)
        return cnt + _first(plsc.all_reduce_population_count(m))

    def cbody(j, cnts):
        cA1, cA2 = cnts
        for u in range(2):
            cA1 = scan_one(0, cA1, j * 2 + u)
            cA2 = scan_one(2, cA2, j * 2 + u)
        return cA1, cA2

    cA1, cA2 = lax.fori_loop(0, _VPC // 2, cbody, (jnp.int32(0),) * 2)

    cnt_v[...] = jnp.where(lane == 0, cA1, 0) + jnp.where(lane == 2, cA2, 0)
    pltpu.sync_copy(cnt_v, counts_sh.at[wid])
    plsc.subcore_barrier()
    pltpu.sync_copy(counts_sh, allc_v)
    base_vec = jnp.zeros((16,), jnp.int32)
    tot_vec = jnp.zeros((16,), jnp.int32)
    for w2 in range(_NW):
        row = allc_v[w2]
        base_vec = base_vec + jnp.where(jnp.int32(w2) < wid, row, 0)
        tot_vec = tot_vec + row
    baseA1 = _lane(base_vec, 0)
    baseA2 = _lane(base_vec, 2)
    tot_fg = _lane(tot_vec, 0)
    tot_bg = _lane(tot_vec, 2)

    def build_pos(s, base, c_self, shift):
        _, cap, out_off = _STREAMS[s]
        for k in range(cap // 128):
            for jj in range(8):
                jvec = k * 128 + jj * 16 + lane
                gpos = shift + base + jvec
                m = (jvec < c_self) & (gpos < cap)
                poss[s][k, pl.ds(jj * 16, 16)] = \
                    jnp.where(m, gpos + out_off, _DUMP)

    def fire_scatter(s):
        _, cap, _ = _STREAMS[s]
        return [pltpu.async_copy(lsts[s].at[pl.ds(k * 128, 128)],
                                 sidx_sh.at[poss[s].at[k]], sem)
                for k in range(cap // 128)]

    build_pos(0, baseA1, cA1, 0)
    build_pos(2, baseA2, cA2, 0)
    descs = fire_scatter(0) + fire_scatter(2)

    def fill_pass(s, lane_id, total):
        cB = lax.fori_loop(0, _VPC, lambda j, c: scan_one(s, c, j),
                           jnp.int32(0))
        plsc.subcore_barrier()
        cnt_v[...] = jnp.where(lane == lane_id, cB, 0)
        pltpu.sync_copy(cnt_v, counts_sh.at[wid])
        plsc.subcore_barrier()
        pltpu.sync_copy(counts_sh, allc_v)
        bv = jnp.zeros((16,), jnp.int32)
        for w2 in range(_NW):
            bv = bv + jnp.where(jnp.int32(w2) < wid, allc_v[w2], 0)
        build_pos(s, _lane(bv, lane_id), cB, total)
        for d in fire_scatter(s):
            d.wait()

    @pl.when(tot_fg < _NUM_FG)
    def _():
        fill_pass(1, 1, tot_fg)

    @pl.when(tot_bg < _NUM_BG)
    def _():
        fill_pass(3, 3, tot_bg)

    for d in descs:
        d.wait()
    plsc.subcore_barrier()

    nslot = (_NUM_FG + _NUM_BG) // _NW
    pltpu.sync_copy(sidx_sh.at[pl.ds(wid * nslot, nslot)], sidx_v)
    sps = (sp0_v, sp1_v, sp2_v, sp3_v)
    sgs = (sg0_v, sg1_v, sg2_v, sg3_v)
    descs = [pltpu.async_copy(vals_hbm.at[sidx_v], sv_v, sem),
             pltpu.async_copy(idxs_hbm.at[sidx_v], mi_v, sem)]
    for c, t in enumerate((pb0_hbm, pb1_hbm, pb2_hbm, pb3_hbm)):
        descs.append(pltpu.async_copy(t.at[sidx_v], sps[c], sem))
    for d in descs:
        d.wait()
    descs = [pltpu.async_copy(gtc_hbm.at[mi_v], sc_v, sem)]
    for c, t in enumerate((gt0_hbm, gt1_hbm, gt2_hbm, gt3_hbm)):
        descs.append(pltpu.async_copy(t.at[mi_v], sgs[c], sem))
    for d in descs:
        d.wait()

    for k in range(nslot // 16):
        rvec = k * 16 + lane
        for c in range(4):
            csplat = jnp.broadcast_to(jnp.int32(c), (16,))
            plsc.store_scatter(outbuf, [rvec, csplat], sps[c][pl.ds(k * 16, 16)])
            plsc.store_scatter(outbuf, [rvec, csplat + 4], sgs[c][pl.ds(k * 16, 16)])
        svv = sv_v[pl.ds(k * 16, 16)]
        plsc.store_scatter(outbuf, [rvec, jnp.broadcast_to(jnp.int32(8), (16,))], svv)
        gtcv = sc_v[pl.ds(k * 16, 16)]
        cls_v[pl.ds(k * 16, 16)] = jnp.where(svv >= 0.5, gtcv, _NUM_CLASSES)

    pltpu.sync_copy(outbuf, out_hbm.at[pl.ds(wid * nslot, nslot)])
    pltpu.sync_copy(cls_v, cls_hbm.at[pl.ds(wid * nslot, nslot)])


def _sc_call():
    mesh = plsc.VectorSubcoreMesh(core_axis_name="c", subcore_axis_name="s",
                                  num_cores=1)
    nslot = (_NUM_FG + _NUM_BG) // _NW
    return pl.kernel(
        _sc_body,
        out_type=[
            jax.ShapeDtypeStruct((_NUM_FG + _NUM_BG, 9), jnp.float32),
            jax.ShapeDtypeStruct((_NUM_FG + _NUM_BG,), jnp.int32),
        ],
        mesh=mesh,
        compiler_params=pltpu.CompilerParams(needs_layout_passes=False,
                                             use_tc_tiling_on_sc=False),
        scratch_types=[
            pltpu.VMEM((_PP,), jnp.int32),
            pltpu.VMEM((_CHUNK,), jnp.int32),
            pltpu.VMEM((_CHUNK,), jnp.int32),
            pltpu.VMEM((_CHUNK,), jnp.int32),
            pltpu.VMEM((_CHUNK,), jnp.int32),
            pltpu.VMEM((_NUM_FG + 16,), jnp.int32),
            pltpu.VMEM((_NUM_FG + 16,), jnp.int32),
            pltpu.VMEM((_NUM_BG + 16,), jnp.int32),
            pltpu.VMEM((_NUM_BG + 16,), jnp.int32),
            pltpu.VMEM((1, 128), jnp.int32),
            pltpu.VMEM((1, 128), jnp.int32),
            pltpu.VMEM((3, 128), jnp.int32),
            pltpu.VMEM((3, 128), jnp.int32),
            pltpu.VMEM((16,), jnp.int32),
            pltpu.VMEM((_NW, 16), jnp.int32),
            pltpu.VMEM((nslot,), jnp.int32),
            pltpu.VMEM((nslot,), jnp.float32),
            pltpu.VMEM((nslot,), jnp.float32),
            pltpu.VMEM((nslot,), jnp.float32),
            pltpu.VMEM((nslot,), jnp.float32),
            pltpu.VMEM((nslot,), jnp.float32),
            pltpu.VMEM((nslot,), jnp.int32),
            pltpu.VMEM((nslot,), jnp.float32),
            pltpu.VMEM((nslot,), jnp.float32),
            pltpu.VMEM((nslot,), jnp.float32),
            pltpu.VMEM((nslot,), jnp.float32),
            pltpu.VMEM((nslot,), jnp.int32),
            pltpu.VMEM((nslot, 9), jnp.float32),
            pltpu.VMEM((nslot,), jnp.int32),
            pltpu.VMEM_SHARED((_NW, 16), jnp.int32),
            pltpu.VMEM_SHARED((_DUMP + 16,), jnp.int32),
            pltpu.SemaphoreType.DMA,
        ],
    )


def kernel(proposal_boxes, gt_boxes, gt_classes):
    perms = [jnp.asarray(p) for p in _perm_consts()]
    pb_pad = jnp.concatenate(
        [proposal_boxes, jnp.zeros((_PP - _P, 4), jnp.float32)], axis=0)
    pbt4 = pb_pad.T
    pbt = pbt4.reshape(4, _ROWS, 128)
    vals, idxs, lab = _iou_call(pbt, gt_boxes)
    gtt = gt_boxes.T
    out, cls = _sc_call()(
        vals.reshape(_PP), idxs.reshape(_PP), lab.reshape(_PP),
        jnp.concatenate(perms),
        pbt4[0], pbt4[1], pbt4[2], pbt4[3],
        gtt[0], gtt[1], gtt[2], gtt[3], gt_classes.astype(jnp.int32))
    return out, cls
